# baseline (speedup 1.0000x reference)
import functools

import jax
import jax.numpy as jnp
from jax import lax
from jax.experimental import pallas as pl
from jax.experimental.pallas import tpu as pltpu
from jax.experimental.pallas import tpu_sc as plsc

N = 10000
E = 320000
D = 128
NUM_GRAPHS = 64

NC = 2
NS = 16
NW = NC * NS
G = 128

N_GROUPS = 160
NPASS = 4
GPP = N_GROUPS // NPASS
EPT = N_GROUPS * G
E_PAD = EPT * NS
NBUF = 2
ROWS_PT = 632
N_ACC = N + 8
TAIL = N - 15 * ROWS_PT
TAIL_Z = N_ACC - N

_mesh = plsc.VectorSubcoreMesh(core_axis_name="c", subcore_axis_name="s")


@functools.partial(
    pl.kernel,
    mesh=_mesh,
    out_type=jax.ShapeDtypeStruct((NC, N, D), jnp.float32),
    scratch_types=[
        pltpu.VMEM((GPP, G), jnp.int32),
        pltpu.VMEM((GPP, G), jnp.int32),
        pltpu.VMEM((NBUF, G, D), jnp.float32),
        pltpu.VMEM_SHARED((N_ACC, D), jnp.float32),
        pltpu.SemaphoreType.DMA,
        pltpu.SemaphoreType.DMA,
    ],
)
def _sc_aggregate(x_hbm, src_hbm, dst_hbm, out_hbm,
                  idx_s, idx_d, rows, acc, sem0, sem1):
    cid = lax.axis_index("c")
    sid = lax.axis_index("s")
    wid = sid
    row0 = sid * ROWS_PT
    xbase = cid * N

    @pl.when(sid < NS - 1)
    def _():
        pltpu.sync_copy(x_hbm.at[pl.ds(xbase + row0, ROWS_PT)],
                        acc.at[pl.ds(row0, ROWS_PT)])

    @pl.when(sid == NS - 1)
    def _():
        pltpu.sync_copy(x_hbm.at[pl.ds(xbase + 15 * ROWS_PT, TAIL)],
                        acc.at[pl.ds(15 * ROWS_PT, TAIL)])

    plsc.subcore_barrier()

    sems = (sem0, sem1)

    @pl.when(cid == 0)
    def _():
        for p in range(NPASS):
            pltpu.sync_copy(src_hbm.at[cid, wid, pl.ds(p * GPP, GPP)], idx_s)
            pltpu.sync_copy(dst_hbm.at[wid, pl.ds(p * GPP, GPP)], idx_d)

            for b in range(NBUF):
                pltpu.async_copy(x_hbm.at[idx_s.at[b]], rows.at[b], sems[b])

            def body(i, carry):
                for b in range(NBUF):
                    g = i * NBUF + b
                    pltpu.make_async_copy(
                        x_hbm.at[idx_s.at[b]], rows.at[b], sems[b]).wait()
                    pltpu.sync_copy(rows.at[b], acc.at[idx_d.at[g]], add=True)

                    @pl.when(g + NBUF < GPP)
                    def _():
                        pltpu.async_copy(
                            x_hbm.at[idx_s.at[g + NBUF]], rows.at[b], sems[b])
                return carry

            lax.fori_loop(0, GPP // NBUF, body, 0)

    plsc.subcore_barrier()

    @pl.when(sid < NS - 1)
    def _():
        pltpu.sync_copy(acc.at[pl.ds(row0, ROWS_PT)],
                        out_hbm.at[cid, pl.ds(row0, ROWS_PT)])

    @pl.when(sid == NS - 1)
    def _():
        pltpu.sync_copy(acc.at[pl.ds(15 * ROWS_PT, TAIL)],
                        out_hbm.at[cid, pl.ds(15 * ROWS_PT, TAIL)])


ROWS_TC = 1000
N_BLOCKS = N // ROWS_TC


def _mlp_body(agg_ref, x_ref, w1_ref, b1_ref, w2_ref, b2_ref):
    h = agg_ref[0] + agg_ref[1] - x_ref[0]
    h = jnp.maximum(
        jnp.dot(h, w1_ref[...], preferred_element_type=jnp.float32)
        + b1_ref[...], 0.0)
    h = jnp.maximum(
        jnp.dot(h, w2_ref[...], preferred_element_type=jnp.float32)
        + b2_ref[...], 0.0)
    return h


def _mlp_kernel(agg_ref, x_ref, w1_ref, b1_ref, w2_ref, b2_ref, o_ref):
    o_ref[0] = _mlp_body(agg_ref, x_ref, w1_ref, b1_ref, w2_ref, b2_ref)


def _mlp_pool_kernel(agg_ref, x_ref, w1_ref, b1_ref, w2_ref, b2_ref,
                     batch_ref, o_ref, counts):
    i = pl.program_id(0)

    @pl.when(i == 0)
    def _():
        o_ref[...] = jnp.zeros_like(o_ref)
        counts[...] = jnp.zeros_like(counts)

    h = _mlp_body(agg_ref, x_ref, w1_ref, b1_ref, w2_ref, b2_ref)
    b = batch_ref[0, 0, :]
    onehot = (b[:, None]
              == lax.broadcasted_iota(jnp.int32, (ROWS_TC, NUM_GRAPHS), 1)
              ).astype(jnp.float32)
    o_ref[...] += lax.dot_general(
        onehot, h, (((0,), (0,)), ((), ())),
        preferred_element_type=jnp.float32)
    counts[...] += jnp.sum(onehot, axis=0)[:, None]

    @pl.when(i == N_BLOCKS - 1)
    def _():
        o_ref[...] = o_ref[...] / jnp.maximum(counts[...], 1.0)


_w_spec = pl.BlockSpec((D, D), lambda i: (0, 0))
_b_spec = pl.BlockSpec((1, D), lambda i: (0, 0))
_agg_spec = pl.BlockSpec((NC, ROWS_TC, D), lambda i: (0, i, 0))


def _tc_mlp(agg, x3, w1, b1, w2, b2):
    return pl.pallas_call(
        _mlp_kernel,
        grid=(NC, N_BLOCKS),
        in_specs=[pl.BlockSpec((NC, ROWS_TC, D), lambda j, i: (0, i, 0)),
                  pl.BlockSpec((1, ROWS_TC, D), lambda j, i: (0, i, 0)),
                  pl.BlockSpec((D, D), lambda j, i: (0, 0)),
                  pl.BlockSpec((1, D), lambda j, i: (0, 0)),
                  pl.BlockSpec((D, D), lambda j, i: (0, 0)),
                  pl.BlockSpec((1, D), lambda j, i: (0, 0))],
        out_specs=pl.BlockSpec((1, ROWS_TC, D), lambda j, i: (j, i, 0)),
        out_shape=jax.ShapeDtypeStruct((NC, N, D), jnp.float32),
    )(agg, x3, w1, b1.reshape(1, D), w2, b2.reshape(1, D))


def _tc_mlp_pool(agg, x3, w1, b1, w2, b2, batch_r):
    return pl.pallas_call(
        _mlp_pool_kernel,
        grid=(N_BLOCKS,),
        in_specs=[_agg_spec,
                  pl.BlockSpec((1, ROWS_TC, D), lambda i: (0, i, 0)),
                  _w_spec, _b_spec, _w_spec, _b_spec,
                  pl.BlockSpec((1, 1, ROWS_TC), lambda i: (i, 0, 0))],
        out_specs=pl.BlockSpec((NUM_GRAPHS, D), lambda i: (0, 0)),
        out_shape=jax.ShapeDtypeStruct((NUM_GRAPHS, D), jnp.float32),
        scratch_shapes=[pltpu.VMEM((NUM_GRAPHS, D), jnp.float32)],
    )(agg, x3, w1, b1.reshape(1, D), w2, b2.reshape(1, D), batch_r)


@jax.jit
def kernel(x, edge_index, batch, W1a, b1a, W2a, b2a, W1b, b1b, W2b, b2b):
    pad = E_PAD - E
    src = jnp.concatenate(
        [edge_index[0], jnp.zeros((pad,), jnp.int32)]).reshape(NS, N_GROUPS, G)
    src = jnp.stack([src, src + N])
    dst = jnp.concatenate(
        [edge_index[1], jnp.full((pad,), N, jnp.int32)]).reshape(NS, N_GROUPS, G)
    batch_r = batch.reshape(N_BLOCKS, 1, ROWS_TC)

    xx = jnp.concatenate([x, x]).reshape(NC * N, D)
    agg1 = _sc_aggregate(xx, src, dst)
    h1 = _tc_mlp(agg1, x.reshape(1, N, D), W1a, b1a, W2a, b2a)
    agg2 = _sc_aggregate(h1.reshape(NC * N, D), src, dst)
    return _tc_mlp_pool(agg2, h1[:1], W1b, b1b, W2b, b2b, batch_r)


# speedup vs baseline: 1.2608x; 1.2608x over previous
import functools

import jax
import jax.numpy as jnp
from jax import lax
from jax.experimental import pallas as pl
from jax.experimental.pallas import tpu as pltpu
from jax.experimental.pallas import tpu_sc as plsc

N = 10000
E = 320000
D = 128
NUM_GRAPHS = 64

NC = 2
NS = 16
NW = NC * NS
G = 128

EPG = NW * G
E_PAD = ((E + EPG - 1) // EPG) * EPG
EPT = E_PAD // NW
N_GROUPS = EPT // G
ROWS_PT = 632
N_ACC = NS * ROWS_PT
TAIL = N - 15 * ROWS_PT
TAIL_Z = N_ACC - N

_mesh = plsc.VectorSubcoreMesh(core_axis_name="c", subcore_axis_name="s")


@functools.partial(
    pl.kernel,
    mesh=_mesh,
    out_type=jax.ShapeDtypeStruct((NC, N, D), jnp.float32),
    scratch_types=[
        pltpu.VMEM((G,), jnp.int32),
        pltpu.VMEM((G,), jnp.int32),
        pltpu.VMEM((G, D), jnp.float32),
        pltpu.VMEM_SHARED((N_ACC, D), jnp.float32),
        pltpu.SemaphoreType.DMA,
    ],
)
def _sc_aggregate(x_hbm, src_hbm, dst_hbm, zrows_hbm, out_hbm,
                  idx_s, idx_d, rows, acc, sem):
    cid = lax.axis_index("c")
    sid = lax.axis_index("s")
    wid = sid * NC + cid
    row0 = sid * ROWS_PT

    @pl.when(jnp.logical_and(cid == 0, sid < NS - 1))
    def _():
        pltpu.sync_copy(x_hbm.at[pl.ds(row0, ROWS_PT)],
                        acc.at[pl.ds(row0, ROWS_PT)])

    @pl.when(jnp.logical_and(cid == 0, sid == NS - 1))
    def _():
        pltpu.sync_copy(x_hbm.at[pl.ds(15 * ROWS_PT, TAIL)],
                        acc.at[pl.ds(15 * ROWS_PT, TAIL)])
        pltpu.sync_copy(zrows_hbm.at[pl.ds(0, TAIL_Z)],
                        acc.at[pl.ds(N, TAIL_Z)])

    @pl.when(cid != 0)
    def _():
        pltpu.sync_copy(zrows_hbm, acc.at[pl.ds(row0, ROWS_PT)])

    plsc.subcore_barrier()

    def body(g, carry):
        base = wid * EPT + g * G
        pltpu.sync_copy(src_hbm.at[pl.ds(base, G)], idx_s)
        pltpu.sync_copy(dst_hbm.at[pl.ds(base, G)], idx_d)
        pltpu.async_copy(x_hbm.at[idx_s], rows, sem).wait()
        pltpu.sync_copy(rows, acc.at[idx_d], add=True)
        return carry

    lax.fori_loop(0, N_GROUPS, body, 0)

    plsc.subcore_barrier()

    @pl.when(sid < NS - 1)
    def _():
        pltpu.sync_copy(acc.at[pl.ds(row0, ROWS_PT)],
                        out_hbm.at[cid, pl.ds(row0, ROWS_PT)])

    @pl.when(sid == NS - 1)
    def _():
        pltpu.sync_copy(acc.at[pl.ds(15 * ROWS_PT, TAIL)],
                        out_hbm.at[cid, pl.ds(15 * ROWS_PT, TAIL)])


ROWS_TC = 1000
N_BLOCKS = N // ROWS_TC


def _mlp_body(agg_ref, w1_ref, b1_ref, w2_ref, b2_ref):
    h = agg_ref[0] + agg_ref[1]
    h = jnp.maximum(
        jnp.dot(h, w1_ref[...], preferred_element_type=jnp.float32)
        + b1_ref[...], 0.0)
    h = jnp.maximum(
        jnp.dot(h, w2_ref[...], preferred_element_type=jnp.float32)
        + b2_ref[...], 0.0)
    return h


def _mlp_kernel(agg_ref, w1_ref, b1_ref, w2_ref, b2_ref, o_ref):
    o_ref[...] = _mlp_body(agg_ref, w1_ref, b1_ref, w2_ref, b2_ref)


def _mlp_pool_kernel(agg_ref, w1_ref, b1_ref, w2_ref, b2_ref, batch_ref,
                     o_ref, counts):
    i = pl.program_id(0)

    @pl.when(i == 0)
    def _():
        o_ref[...] = jnp.zeros_like(o_ref)
        counts[...] = jnp.zeros_like(counts)

    h = _mlp_body(agg_ref, w1_ref, b1_ref, w2_ref, b2_ref)
    b = batch_ref[0, 0, :]
    onehot = (b[:, None]
              == lax.broadcasted_iota(jnp.int32, (ROWS_TC, NUM_GRAPHS), 1)
              ).astype(jnp.float32)
    o_ref[...] += lax.dot_general(
        onehot, h, (((0,), (0,)), ((), ())),
        preferred_element_type=jnp.float32)
    counts[...] += jnp.sum(onehot, axis=0)[:, None]

    @pl.when(i == N_BLOCKS - 1)
    def _():
        o_ref[...] = o_ref[...] / jnp.maximum(counts[...], 1.0)


_w_spec = pl.BlockSpec((D, D), lambda i: (0, 0))
_b_spec = pl.BlockSpec((1, D), lambda i: (0, 0))
_agg_spec = pl.BlockSpec((NC, ROWS_TC, D), lambda i: (0, i, 0))


def _tc_mlp(agg, w1, b1, w2, b2):
    return pl.pallas_call(
        _mlp_kernel,
        grid=(N_BLOCKS,),
        in_specs=[_agg_spec, _w_spec, _b_spec, _w_spec, _b_spec],
        out_specs=pl.BlockSpec((ROWS_TC, D), lambda i: (i, 0)),
        out_shape=jax.ShapeDtypeStruct((N, D), jnp.float32),
    )(agg, w1, b1.reshape(1, D), w2, b2.reshape(1, D))


def _tc_mlp_pool(agg, w1, b1, w2, b2, batch_r):
    return pl.pallas_call(
        _mlp_pool_kernel,
        grid=(N_BLOCKS,),
        in_specs=[_agg_spec, _w_spec, _b_spec, _w_spec, _b_spec,
                  pl.BlockSpec((1, 1, ROWS_TC), lambda i: (i, 0, 0))],
        out_specs=pl.BlockSpec((NUM_GRAPHS, D), lambda i: (0, 0)),
        out_shape=jax.ShapeDtypeStruct((NUM_GRAPHS, D), jnp.float32),
        scratch_shapes=[pltpu.VMEM((NUM_GRAPHS, D), jnp.float32)],
    )(agg, w1, b1.reshape(1, D), w2, b2.reshape(1, D), batch_r)


@jax.jit
def kernel(x, edge_index, batch, W1a, b1a, W2a, b2a, W1b, b1b, W2b, b2b):
    pad = E_PAD - E
    src = jnp.concatenate([edge_index[0], jnp.zeros((pad,), jnp.int32)])
    dst = jnp.concatenate([edge_index[1], jnp.full((pad,), N, jnp.int32)])
    zrows = jnp.zeros((ROWS_PT, D), jnp.float32)
    batch_r = batch.reshape(N_BLOCKS, 1, ROWS_TC)

    agg1 = _sc_aggregate(x, src, dst, zrows)
    h1 = _tc_mlp(agg1, W1a, b1a, W2a, b2a)
    agg2 = _sc_aggregate(h1, src, dst, zrows)
    return _tc_mlp_pool(agg2, W1b, b1b, W2b, b2b, batch_r)
